# baseline (speedup 1.0000x reference)
import jax
import jax.numpy as jnp
from jax import lax
from jax.experimental import pallas as pl
from jax.experimental.pallas import tpu as pltpu
from jax.experimental.pallas import tpu_sc as plsc

Q = 1024
D = 16
N = 100000
NCLS = 1000
KNN = 8

CHUNK = 1024
NPAD = 100352
NCHUNKS = NPAD // CHUNK
GROUP = 8
CAPQ = 144
CPQ = CAPQ + 16
NCVREG = CPQ // 16

NW = 32
QPW = Q // NW

F32MAX = 3.4e38
IMAX = 2**31 - 1


def _tc_dist_body(x_ref, xt_ref, tn_ref, out_ref):
    d = lax.dot_general(x_ref[...], xt_ref[...], (((1,), (1,)), ((), ())),
                        preferred_element_type=jnp.float32)
    s = tn_ref[...] - 2.0 * d
    out_ref[...] = s.reshape(1, NW, QPW, 1024)


def _tc_dists(x, xp, tn2):
    return pl.pallas_call(
        _tc_dist_body,
        grid=(NPAD // 1024,),
        in_specs=[
            pl.BlockSpec((Q, D), lambda i: (0, 0)),
            pl.BlockSpec((1024, D), lambda i: (i, 0)),
            pl.BlockSpec((1, 1024), lambda i: (0, i)),
        ],
        out_specs=pl.BlockSpec((1, NW, QPW, 1024), lambda i: (i, 0, 0, 0)),
        out_shape=jax.ShapeDtypeStruct((NCHUNKS, NW, QPW, 1024), jnp.float32),
    )(x, xp, tn2)


def _top8_of_cands(cand_v, cand_i, base):
    lanes = lax.iota(jnp.int32, 16)

    def pass_body(k, st):
        win_v, win_i, pv, pi = st

        def vreg_body(j, bst):
            best, besti = bst
            v = cand_v[pl.ds(base + j * 16, 16)]
            iv = cand_i[pl.ds(base + j * 16, 16)]
            valid = (v > pv) | ((v == pv) & (iv > pi))
            v2 = jnp.where(valid, v, F32MAX)
            iv2 = jnp.where(valid, iv, IMAX)
            better = (v2 < best) | ((v2 == best) & (iv2 < besti))
            return (jnp.where(better, v2, best), jnp.where(better, iv2, besti))

        best, besti = lax.fori_loop(
            0, NCVREG, vreg_body,
            (jnp.full((16,), F32MAX, jnp.float32),
             jnp.full((16,), IMAX, jnp.int32)))
        s = jnp.min(best)
        iw = jnp.min(jnp.where(best == s, besti, IMAX))
        win_v = jnp.where(lanes == k, s, win_v)
        win_i = jnp.where(lanes == k, iw, win_i)
        return (win_v, win_i, s, iw)

    init = (jnp.full((16,), F32MAX, jnp.float32),
            jnp.full((16,), IMAX, jnp.int32),
            jnp.float32(-F32MAX), jnp.int32(-2**31))
    win_v, win_i, _, _ = lax.fori_loop(0, KNN, pass_body, init)
    return win_v, win_i


def _reset_cands(cand_v, cand_i, base, win_v, win_i):
    cand_v[pl.ds(base, 16)] = win_v
    cand_i[pl.ds(base, 16)] = win_i
    fill_v = jnp.full((16,), F32MAX, jnp.float32)
    fill_i = jnp.full((16,), IMAX, jnp.int32)

    def fill(j, _):
        cand_v[pl.ds(base + j * 16, 16)] = fill_v
        cand_i[pl.ds(base + j * 16, 16)] = fill_i
        return 0

    lax.fori_loop(1, NCVREG, fill, 0)


def _sc_body(s_hbm, x_hbm, y_hbm, out_hbm,
             sbufa, sbufb, cand_v, cand_i, logits_l, xq, idxbuf, ybuf,
             tst, cst, sema, semb, gsem):
    cid = lax.axis_index("c")
    sid = lax.axis_index("s")
    wid = sid * 2 + cid
    qbase = wid * QPW
    lanes = lax.iota(jnp.int32, 16)
    zeros16 = jnp.zeros((16,), jnp.float32)
    fill_v = jnp.full((16,), F32MAX, jnp.float32)
    fill_i = jnp.full((16,), IMAX, jnp.int32)

    pltpu.sync_copy(x_hbm.at[pl.ds(qbase, QPW), :], xq)

    def zrow(r, _):
        def zcol(c, _2):
            logits_l[r, pl.ds(c * 16, 16)] = zeros16
            return 0
        lax.fori_loop(0, 1024 // 16, zcol, 0)
        return 0
    lax.fori_loop(0, QPW, zrow, 0)

    def cfill(j, _):
        cand_v[pl.ds(j * 16, 16)] = fill_v
        cand_i[pl.ds(j * 16, 16)] = fill_i
        return 0
    lax.fori_loop(0, QPW * NCVREG, cfill, 0)

    def sinit(ql, _):
        tst[ql] = jnp.float32(F32MAX)
        cst[ql] = jnp.int32(0)
        return 0
    lax.fori_loop(0, QPW, sinit, 0)

    def scan_block(buf, cbase):
        def per_q(ql, _):
            qb = ql * CPQ
            t0 = tst[ql]
            tv0 = jnp.full((16,), t0)

            nacc = 8
            accs = [buf[ql, pl.ds(j * 16, 16)] < tv0 for j in range(nacc)]
            for j in range(nacc, CHUNK // 16):
                accs[j % nacc] = accs[j % nacc] | (
                    buf[ql, pl.ds(j * 16, 16)] < tv0)
            rowhit = accs[0]
            for a in accs[1:]:
                rowhit = rowhit | a

            def per_group(g, gst):
                t, cnt = gst
                tv = jnp.full((16,), t)
                vs = [buf[ql, pl.ds(g * (GROUP * 16) + j * 16, 16)]
                      for j in range(GROUP)]
                ms = [v < tv for v in vs]
                hit = ms[0]
                for m in ms[1:]:
                    hit = hit | m
                anyhit = jnp.any(hit)

                def do_append(cnt):
                    pcs = [plsc.all_reduce_population_count(ms[j])[0]
                           for j in range(GROUP)]
                    off = cnt
                    for j in range(GROUP):
                        iv = (cbase + g * (GROUP * 16) + j * 16) + lanes
                        plsc.store_compressed(
                            cand_v.at[pl.ds(qb + off, 16)], vs[j], mask=ms[j])
                        plsc.store_compressed(
                            cand_i.at[pl.ds(qb + off, 16)], iv, mask=ms[j])
                        off = off + pcs[j]
                    return off

                cnt = lax.cond(anyhit, do_append, lambda c: c, cnt)

                def do_rebuild(_):
                    wv, wi = _top8_of_cands(cand_v, cand_i, qb)
                    _reset_cands(cand_v, cand_i, qb, wv, wi)
                    return (wv[KNN - 1], jnp.int32(16))

                t, cnt = lax.cond(cnt >= CAPQ - GROUP * 16,
                                  do_rebuild, lambda _: (t, cnt), 0)
                return (t, cnt)

            @pl.when(jnp.any(rowhit))
            def _():
                t, cnt = lax.fori_loop(0, CHUNK // (GROUP * 16), per_group,
                                       (t0, cst[ql]))
                tst[ql] = t
                cst[ql] = cnt
            return 0

        lax.fori_loop(0, QPW, per_q, 0)

    pltpu.async_copy(s_hbm.at[0, wid], sbufa, sema)
    pltpu.async_copy(s_hbm.at[1, wid], sbufb, semb)

    def per_pair(cp, _):
        c0 = 2 * cp

        pltpu.make_async_copy(s_hbm.at[c0, wid], sbufa, sema).wait()
        scan_block(sbufa, c0 * CHUNK)

        @pl.when(c0 + 2 < NCHUNKS)
        def _():
            pltpu.async_copy(s_hbm.at[c0 + 2, wid], sbufa, sema)

        pltpu.make_async_copy(s_hbm.at[c0 + 1, wid], sbufb, semb).wait()
        scan_block(sbufb, (c0 + 1) * CHUNK)

        @pl.when(c0 + 3 < NCHUNKS)
        def _():
            pltpu.async_copy(s_hbm.at[c0 + 3, wid], sbufb, semb)

        return 0

    lax.fori_loop(0, NCHUNKS // 2, per_pair, 0)

    def finalize(ql, _):
        qb = ql * CPQ
        win_v, win_i = _top8_of_cands(cand_v, cand_i, qb)

        qv = xq[ql]
        xn = jnp.sum(qv * qv)
        w = 1.0 / jnp.maximum(win_v + xn, jnp.float32(1e-6))

        kmask = lanes < KNN
        idxbuf[...] = jnp.where(kmask, win_i, 0)
        pltpu.async_copy(y_hbm.at[idxbuf], ybuf, gsem).wait()
        labels = ybuf[...]

        row = jnp.full((16,), ql, jnp.int32)
        for k in range(KNN):
            plsc.addupdate_scatter(logits_l, [row, labels], w,
                                   mask=lanes == k)
        return 0

    lax.fori_loop(0, QPW, finalize, 0)

    pltpu.sync_copy(logits_l, out_hbm.at[pl.ds(qbase, QPW), :])


@jax.jit
def kernel(x, Xtrain, ytrain):
    pad = jnp.zeros((NPAD - N, D), jnp.float32).at[:, 0].set(1e18)
    xp = jnp.concatenate([Xtrain.astype(jnp.float32), pad], axis=0)
    tn2 = jnp.sum(xp * xp, axis=1).reshape(1, NPAD)
    s = _tc_dists(x.astype(jnp.float32), xp, tn2)

    mesh = plsc.VectorSubcoreMesh(core_axis_name="c", subcore_axis_name="s")
    sc = pl.kernel(
        _sc_body,
        out_type=jax.ShapeDtypeStruct((Q, 1024), jnp.float32),
        mesh=mesh,
        compiler_params=pltpu.CompilerParams(needs_layout_passes=False),
        scratch_types=[
            pltpu.VMEM((QPW, CHUNK), jnp.float32),
            pltpu.VMEM((QPW, CHUNK), jnp.float32),
            pltpu.VMEM((QPW * CPQ,), jnp.float32),
            pltpu.VMEM((QPW * CPQ,), jnp.int32),
            pltpu.VMEM((QPW, 1024), jnp.float32),
            pltpu.VMEM((QPW, D), jnp.float32),
            pltpu.VMEM((16,), jnp.int32),
            pltpu.VMEM((16,), jnp.int32),
            pltpu.SMEM((QPW,), jnp.float32),
            pltpu.SMEM((QPW,), jnp.int32),
            pltpu.SemaphoreType.DMA,
            pltpu.SemaphoreType.DMA,
            pltpu.SemaphoreType.DMA,
        ],
    )
    out = sc(s, x.astype(jnp.float32), ytrain.astype(jnp.int32))
    return out[:, :NCLS]


# speedup vs baseline: 4.1243x; 4.1243x over previous
import jax
import jax.numpy as jnp
from jax import lax
from jax.experimental import pallas as pl
from jax.experimental.pallas import tpu as pltpu
from jax.experimental.pallas import tpu_sc as plsc

Q = 1024
D = 16
N = 100000
NCLS = 1000
KNN = 8

CHUNK = 1024
NPAD = 100352
NCHUNKS = NPAD // CHUNK
GROUP = 8
CAPQ = 176
CPQ = CAPQ + 16
NCVREG = CPQ // 16

NW = 32
QPW = Q // NW

F32MAX = 3.4e38
IMAX = 2**31 - 1


def _tc_dist_body(x_ref, xt_ref, tn_ref, out_ref):
    d = lax.dot_general(x_ref[...], xt_ref[...], (((1,), (1,)), ((), ())),
                        preferred_element_type=jnp.float32)
    s = tn_ref[...] - 2.0 * d
    out_ref[...] = s.reshape(1, NW, QPW, 1024)


def _tc_dists(x, xp, tn2):
    return pl.pallas_call(
        _tc_dist_body,
        grid=(NPAD // 1024,),
        in_specs=[
            pl.BlockSpec((Q, D), lambda i: (0, 0)),
            pl.BlockSpec((1024, D), lambda i: (i, 0)),
            pl.BlockSpec((1, 1024), lambda i: (0, i)),
        ],
        out_specs=pl.BlockSpec((1, NW, QPW, 1024), lambda i: (i, 0, 0, 0)),
        out_shape=jax.ShapeDtypeStruct((NCHUNKS, NW, QPW, 1024), jnp.float32),
    )(x, xp, tn2)


def _top8_of_cands(cand_v, cand_i, base):
    lanes = lax.iota(jnp.int32, 16)

    def pass_body(k, st):
        win_v, win_i, pv, pi = st

        def vreg_body(j, bst):
            best, besti = bst
            v = cand_v[pl.ds(base + j * 16, 16)]
            iv = cand_i[pl.ds(base + j * 16, 16)]
            valid = (v > pv) | ((v == pv) & (iv > pi))
            v2 = jnp.where(valid, v, F32MAX)
            iv2 = jnp.where(valid, iv, IMAX)
            better = (v2 < best) | ((v2 == best) & (iv2 < besti))
            return (jnp.where(better, v2, best), jnp.where(better, iv2, besti))

        best, besti = lax.fori_loop(
            0, NCVREG, vreg_body,
            (jnp.full((16,), F32MAX, jnp.float32),
             jnp.full((16,), IMAX, jnp.int32)))
        s = jnp.min(best)
        iw = jnp.min(jnp.where(best == s, besti, IMAX))
        win_v = jnp.where(lanes == k, s, win_v)
        win_i = jnp.where(lanes == k, iw, win_i)
        return (win_v, win_i, s, iw)

    init = (jnp.full((16,), F32MAX, jnp.float32),
            jnp.full((16,), IMAX, jnp.int32),
            jnp.float32(-F32MAX), jnp.int32(-2**31))
    win_v, win_i, _, _ = lax.fori_loop(0, KNN, pass_body, init)
    return win_v, win_i


def _reset_cands(cand_v, cand_i, base, win_v, win_i):
    cand_v[pl.ds(base, 16)] = win_v
    cand_i[pl.ds(base, 16)] = win_i
    fill_v = jnp.full((16,), F32MAX, jnp.float32)
    fill_i = jnp.full((16,), IMAX, jnp.int32)

    def fill(j, _):
        cand_v[pl.ds(base + j * 16, 16)] = fill_v
        cand_i[pl.ds(base + j * 16, 16)] = fill_i
        return 0

    lax.fori_loop(1, NCVREG, fill, 0)


def _sc_body(s_hbm, x_hbm, y_hbm, out_hbm,
             sbufa, sbufb, cand_v, cand_i, logits_l, xq, idxbuf, ybuf,
             tst, cst, sema, semb, gsem):
    cid = lax.axis_index("c")
    sid = lax.axis_index("s")
    wid = sid * 2 + cid
    qbase = wid * QPW
    lanes = lax.iota(jnp.int32, 16)
    zeros16 = jnp.zeros((16,), jnp.float32)
    fill_v = jnp.full((16,), F32MAX, jnp.float32)
    fill_i = jnp.full((16,), IMAX, jnp.int32)

    pltpu.sync_copy(x_hbm.at[pl.ds(qbase, QPW), :], xq)

    def zrow(r, _):
        def zcol(c, _2):
            logits_l[r, pl.ds(c * 16, 16)] = zeros16
            return 0
        lax.fori_loop(0, 1024 // 16, zcol, 0)
        return 0
    lax.fori_loop(0, QPW, zrow, 0)

    def cfill(j, _):
        cand_v[pl.ds(j * 16, 16)] = fill_v
        cand_i[pl.ds(j * 16, 16)] = fill_i
        return 0
    lax.fori_loop(0, QPW * NCVREG, cfill, 0)

    def sinit(ql, _):
        tst[ql] = jnp.float32(F32MAX)
        cst[ql] = jnp.int32(0)
        return 0
    lax.fori_loop(0, QPW, sinit, 0)

    def scan_block(buf, cbase):
        def per_q(ql, _):
            qb = ql * CPQ
            t0 = tst[ql]
            tv0 = jnp.full((16,), t0)

            nacc = 8
            accs = [buf[ql, pl.ds(j * 16, 16)] < tv0 for j in range(nacc)]
            for j in range(nacc, CHUNK // 16):
                accs[j % nacc] = accs[j % nacc] | (
                    buf[ql, pl.ds(j * 16, 16)] < tv0)
            rowhit = accs[0]
            for a in accs[1:]:
                rowhit = rowhit | a

            def per_group(g, gst):
                t, cnt = gst
                tv = jnp.full((16,), t)
                vs = [buf[ql, pl.ds(g * (GROUP * 16) + j * 16, 16)]
                      for j in range(GROUP)]
                ms = [v < tv for v in vs]
                hit = ms[0]
                for m in ms[1:]:
                    hit = hit | m
                anyhit = jnp.any(hit)

                def do_append(cnt):
                    pcs = [plsc.all_reduce_population_count(ms[j])[0]
                           for j in range(GROUP)]
                    off = cnt
                    for j in range(GROUP):
                        iv = (cbase + g * (GROUP * 16) + j * 16) + lanes
                        plsc.store_compressed(
                            cand_v.at[pl.ds(qb + off, 16)], vs[j], mask=ms[j])
                        plsc.store_compressed(
                            cand_i.at[pl.ds(qb + off, 16)], iv, mask=ms[j])
                        off = off + pcs[j]
                    return off

                cnt = lax.cond(anyhit, do_append, lambda c: c, cnt)

                def do_rebuild(_):
                    wv, wi = _top8_of_cands(cand_v, cand_i, qb)
                    _reset_cands(cand_v, cand_i, qb, wv, wi)
                    return (wv[KNN - 1], jnp.int32(16))

                t, cnt = lax.cond(cnt >= CAPQ - GROUP * 16,
                                  do_rebuild, lambda _: (t, cnt), 0)
                return (t, cnt)

            @pl.when(jnp.any(rowhit))
            def _():
                t, cnt = lax.fori_loop(0, CHUNK // (GROUP * 16), per_group,
                                       (t0, cst[ql]))
                tst[ql] = t
                cst[ql] = cnt
            return 0

        lax.fori_loop(0, QPW, per_q, 0)

    pltpu.async_copy(s_hbm.at[0, wid], sbufa, sema)
    pltpu.async_copy(s_hbm.at[1, wid], sbufb, semb)

    def per_pair(cp, _):
        c0 = 2 * cp

        pltpu.make_async_copy(s_hbm.at[c0, wid], sbufa, sema).wait()
        scan_block(sbufa, c0 * CHUNK)

        @pl.when(c0 + 2 < NCHUNKS)
        def _():
            pltpu.async_copy(s_hbm.at[c0 + 2, wid], sbufa, sema)

        pltpu.make_async_copy(s_hbm.at[c0 + 1, wid], sbufb, semb).wait()
        scan_block(sbufb, (c0 + 1) * CHUNK)

        @pl.when(c0 + 3 < NCHUNKS)
        def _():
            pltpu.async_copy(s_hbm.at[c0 + 3, wid], sbufb, semb)

        return 0

    lax.fori_loop(0, NCHUNKS // 2, per_pair, 0)

    def finalize(ql, _):
        qb = ql * CPQ
        win_v, win_i = _top8_of_cands(cand_v, cand_i, qb)

        qv = xq[ql]
        xn = jnp.sum(qv * qv)
        w = 1.0 / jnp.maximum(win_v + xn, jnp.float32(1e-6))

        kmask = lanes < KNN
        idxbuf[...] = jnp.where(kmask, win_i, 0)
        pltpu.async_copy(y_hbm.at[idxbuf], ybuf, gsem).wait()
        labels = ybuf[...]

        row = jnp.full((16,), ql, jnp.int32)
        for k in range(KNN):
            plsc.addupdate_scatter(logits_l, [row, labels], w,
                                   mask=lanes == k)
        return 0

    lax.fori_loop(0, QPW, finalize, 0)

    pltpu.sync_copy(logits_l, out_hbm.at[pl.ds(qbase, QPW), :])


@jax.jit
def kernel(x, Xtrain, ytrain):
    pad = jnp.zeros((NPAD - N, D), jnp.float32).at[:, 0].set(1e18)
    xp = jnp.concatenate([Xtrain.astype(jnp.float32), pad], axis=0)
    tn2 = jnp.sum(xp * xp, axis=1).reshape(1, NPAD)
    s = _tc_dists(x.astype(jnp.float32), xp, tn2)

    mesh = plsc.VectorSubcoreMesh(core_axis_name="c", subcore_axis_name="s")
    sc = pl.kernel(
        _sc_body,
        out_type=jax.ShapeDtypeStruct((Q, 1024), jnp.float32),
        mesh=mesh,
        compiler_params=pltpu.CompilerParams(needs_layout_passes=False),
        scratch_types=[
            pltpu.VMEM((QPW, CHUNK), jnp.float32),
            pltpu.VMEM((QPW, CHUNK), jnp.float32),
            pltpu.VMEM((QPW * CPQ,), jnp.float32),
            pltpu.VMEM((QPW * CPQ,), jnp.int32),
            pltpu.VMEM((QPW, 1024), jnp.float32),
            pltpu.VMEM((QPW, D), jnp.float32),
            pltpu.VMEM((16,), jnp.int32),
            pltpu.VMEM((16,), jnp.int32),
            pltpu.SMEM((QPW,), jnp.float32),
            pltpu.SMEM((QPW,), jnp.int32),
            pltpu.SemaphoreType.DMA,
            pltpu.SemaphoreType.DMA,
            pltpu.SemaphoreType.DMA,
        ],
    )
    out = sc(s, x.astype(jnp.float32), ytrain.astype(jnp.int32))
    return out[:, :NCLS]
